# baseline (speedup 1.0000x reference)
import functools

import jax
import jax.numpy as jnp
from jax import lax
from jax.experimental import pallas as pl
from jax.experimental.pallas import tpu as pltpu
from jax.experimental.pallas import tpu_sc as plsc

_VOCAB = 1000
_VPAD = 1024
_HIDDEN = 128
_BATCH = 4096
_SEQ = 20
_SEQP = 24

_NC = 2
_NS = 16
_NW = _NC * _NS
_BPW = _BATCH // _NW
_NBUF = 4
_NGRP = _BPW // _NBUF


def _table_body(emb_ref, w_ref, b_ref, out_ref):
    out_ref[...] = lax.dot_general(
        emb_ref[...], w_ref[...],
        (((1,), (1,)), ((), ())),
        preferred_element_type=jnp.float32,
        precision=lax.Precision.HIGHEST,
    ) + b_ref[...]


def _compute_table(emb, w, b):
    return pl.pallas_call(
        _table_body,
        out_shape=jax.ShapeDtypeStruct((_VOCAB, _VPAD), jnp.float32),
    )(emb, w, b.reshape(1, _VPAD))


_mesh = plsc.VectorSubcoreMesh(core_axis_name="c", subcore_axis_name="s")


@functools.partial(
    pl.kernel,
    mesh=_mesh,
    compiler_params=pltpu.CompilerParams(disable_bounds_checks=True),
    out_type=jax.ShapeDtypeStruct((_BATCH, _SEQ, _VOCAB), jnp.float32),
    scratch_types=[
        pltpu.VMEM((_BPW * _SEQP,), jnp.int32),
        pltpu.VMEM((_NBUF, _SEQP, _VPAD), jnp.float32),
        pltpu.SemaphoreType.DMA((_NBUF,)),
        pltpu.SemaphoreType.DMA((_NBUF,)),
    ],
)
def _gather(table_hbm, idx_hbm, out_hbm, idx_v, rows_v, gsem, wsem):
    wid = lax.axis_index("s") * _NC + lax.axis_index("c")
    base = wid * _BPW
    pltpu.sync_copy(idx_hbm.at[pl.ds(base * _SEQP, _BPW * _SEQP)], idx_v)

    def _gather_start(c, p):
        off = pl.multiple_of(c * _SEQP, 8)
        pltpu.async_copy(
            table_hbm.at[idx_v.at[pl.ds(off, _SEQP)]], rows_v.at[p], gsem.at[p]
        )

    def _gather_wait(p):
        pltpu.make_async_copy(
            table_hbm.at[idx_v.at[pl.ds(0, _SEQP)]], rows_v.at[p], gsem.at[p]
        ).wait()

    def _write_start(c, p):
        pltpu.async_copy(
            rows_v.at[p],
            out_hbm.at[base + c, pl.ds(0, _SEQP), pl.ds(0, _VPAD)],
            wsem.at[p],
        )

    def _write_wait(p):
        pltpu.make_async_copy(
            rows_v.at[p],
            out_hbm.at[base, pl.ds(0, _SEQP), pl.ds(0, _VPAD)],
            wsem.at[p],
        ).wait()

    for p in range(_NBUF):
        _gather_start(p, p)

    def body(g, carry):
        for p in range(_NBUF):
            c = g * _NBUF + p
            _gather_wait(p)
            _write_start(c, p)

            @pl.when(g < _NGRP - 1)
            def _():
                _write_wait(p)
                _gather_start(c + _NBUF, p)

        return carry

    lax.fori_loop(0, _NGRP, body, 0)
    for p in range(_NBUF):
        _write_wait(p)


def kernel(input_ids, embed_table, head_w, head_b):
    w_pad = jnp.pad(head_w, ((0, _VPAD - _VOCAB), (0, 0)))
    b_pad = jnp.pad(head_b, (0, _VPAD - _VOCAB))
    table = _compute_table(embed_table, w_pad, b_pad)
    idx = jnp.pad(input_ids.astype(jnp.int32), ((0, 0), (0, _SEQP - _SEQ)))
    out = _gather(table, idx.reshape(-1))
    return out


# speedup vs baseline: 1.0035x; 1.0035x over previous
import functools

import jax
import jax.numpy as jnp
from jax import lax
from jax.experimental import pallas as pl
from jax.experimental.pallas import tpu as pltpu
from jax.experimental.pallas import tpu_sc as plsc

_VOCAB = 1000
_VPAD = 1024
_NT = _VPAD // 128
_HIDDEN = 128
_BATCH = 4096
_SEQ = 20
_SEQP = 24
_NST = _SEQP // 8

_NC = 2
_NS = 16
_NW = _NC * _NS
_BPW = _BATCH // _NW
_NGRP = _BPW // 2


def _table_body(emb_ref, w_ref, b_ref, out_ref):
    out_ref[...] = lax.dot_general(
        emb_ref[...], w_ref[...],
        (((1,), (1,)), ((), ())),
        preferred_element_type=jnp.float32,
        precision=lax.Precision.HIGHEST,
    ) + b_ref[...]


def _compute_table(emb, w, b):
    return pl.pallas_call(
        _table_body,
        out_shape=jax.ShapeDtypeStruct((_VOCAB, _VPAD), jnp.float32),
    )(emb, w, b.reshape(1, _VPAD))


_mesh = plsc.VectorSubcoreMesh(core_axis_name="c", subcore_axis_name="s")


@functools.partial(
    pl.kernel,
    mesh=_mesh,
    compiler_params=pltpu.CompilerParams(disable_bounds_checks=True),
    out_type=jax.ShapeDtypeStruct((_BATCH, _SEQ, _VOCAB), jnp.float32),
    scratch_types=[
        pltpu.VMEM((_BPW * _SEQP,), jnp.int32),
        pltpu.VMEM((2, _SEQP, _NT, 128), jnp.float32),
        pltpu.VMEM((2, _SEQP, _VPAD), jnp.float32),
        pltpu.SemaphoreType.DMA((2,)),
        pltpu.SemaphoreType.DMA((2,)),
    ],
)
def _gather(table_hbm, idx_hbm, out_hbm, idx_v, buf1, buf2, gsem, wsem):
    wid = lax.axis_index("s") * _NC + lax.axis_index("c")
    base = wid * _BPW
    pltpu.sync_copy(idx_hbm.at[pl.ds(base * _SEQP, _BPW * _SEQP)], idx_v)

    def _gather_start(c, p):
        off = pl.multiple_of(c * _SEQP, 8)
        pltpu.async_copy(
            table_hbm.at[idx_v.at[pl.ds(off, _SEQP)]], buf1.at[p], gsem.at[p]
        )

    def _gather_wait(p):
        pltpu.make_async_copy(
            table_hbm.at[idx_v.at[pl.ds(0, _SEQP)]], buf1.at[p], gsem.at[p]
        ).wait()

    def _write_start(c, p):
        pltpu.async_copy(
            buf2.at[p],
            out_hbm.at[base + c, pl.ds(0, _SEQP), pl.ds(0, _VPAD)],
            wsem.at[p],
        )

    def _write_wait(p):
        pltpu.make_async_copy(
            buf2.at[p],
            out_hbm.at[base, pl.ds(0, _SEQP), pl.ds(0, _VPAD)],
            wsem.at[p],
        ).wait()

    def _repack(p):
        def rbody(r, carry):
            for cg in range(_NT):
                for k in range(8):
                    buf2[p, r, pl.ds(128 * cg + 16 * k, 16)] = buf1[
                        p, r, cg, pl.ds(16 * k, 16)
                    ]
            return carry

        lax.fori_loop(0, _SEQP, rbody, 0)

    _gather_start(0, 0)
    _gather_start(1, 1)

    def body(g, carry):
        for p in range(2):
            c = 2 * g + p
            _gather_wait(p)

            @pl.when(c > 1)
            def _():
                _write_wait(p)

            _repack(p)
            _write_start(c, p)

            @pl.when(c < _BPW - 2)
            def _():
                _gather_start(c + 2, p)

        return carry

    lax.fori_loop(0, _NGRP, body, 0)
    for p in range(2):
        _write_wait(p)


def kernel(input_ids, embed_table, head_w, head_b):
    w_pad = jnp.pad(head_w, ((0, _VPAD - _VOCAB), (0, 0)))
    b_pad = jnp.pad(head_b, (0, _VPAD - _VOCAB))
    table = _compute_table(embed_table, w_pad, b_pad)
    table3 = table.reshape(_VOCAB, _NT, 128)
    idx = jnp.pad(input_ids.astype(jnp.int32), ((0, 0), (0, _SEQP - _SEQ)))
    out = _gather(table3, idx.reshape(-1))
    return out
